# baseline (speedup 1.0000x reference)
import jax
import jax.numpy as jnp
from jax import lax
from jax.experimental import pallas as pl

NUM_LEVELS = 3
THRESHOLD = 0.1


def _ge_kernel(g_ref, weight_ref, bias_ref, wq_ref, bq_ref, wk_ref, bk_ref,
               wa_ref, ba_ref, out_ref):
    b, d, n = g_ref.shape
    bias_col = bias_ref[...]
    wa = wa_ref[...]
    wa_q = wa[:, :d]
    wa_k = wa[:, d:]
    vq = jnp.dot(wa_q, wq_ref[...], preferred_element_type=jnp.float32)
    vk = jnp.dot(wa_k, wk_ref[...], preferred_element_type=jnp.float32)
    cq = jnp.sum(bq_ref[...] * wa_q)
    ck = jnp.sum(bk_ref[...] * wa_k)
    const = cq + ck + ba_ref[0, 0]
    w_ext = jnp.concatenate([weight_ref[...].T, vq, vk], axis=0)
    ones_row = jnp.ones((1, n), dtype=jnp.float32)

    row = lax.broadcasted_iota(jnp.int32, (1, n, n), 1)
    col = lax.broadcasted_iota(jnp.int32, (1, n, n), 2)
    offdiag = row != col

    gs = [g_ref[i] for i in range(b)]
    for _ in range(NUM_LEVELS):
        hws = [jnp.dot(w_ext, g, preferred_element_type=jnp.float32)
               for g in gs]
        xwts = [hw[:d, :] for hw in hws]
        ss = jnp.stack([hw[d:d + 2, :] for hw in hws])
        sq_row = ss[:, 0:1, :]
        sk_row = ss[:, 1:2, :]
        sq_col = jnp.transpose(sq_row, (0, 2, 1))
        scores = sq_col + sk_row + const
        probs = jax.nn.sigmoid(scores)
        w_edge = jnp.where(offdiag & (probs > THRESHOLD), probs, 0.0)
        deg = jnp.stack([jnp.dot(ones_row, w_edge[i],
                                 preferred_element_type=jnp.float32)
                         for i in range(b)])
        dinv_row = jnp.where(deg > 0, lax.rsqrt(deg), 0.0)
        dinv_col = jnp.transpose(dinv_row, (0, 2, 1))
        w_norm = dinv_col * w_edge * dinv_row
        gs = [jnp.dot(xwts[i], w_norm[i],
                      preferred_element_type=jnp.float32) + bias_col
              for i in range(b)]
    for i in range(b):
        out_ref[i] = gs[i]


def kernel(x, weight, bias, Wq, bq, Wk, bk, Wa, ba):
    b, d, n = x.shape[0], x.shape[1], x.shape[2]
    bias2 = bias.reshape(d, 1)
    bq2 = bq.reshape(1, d)
    bk2 = bk.reshape(1, d)
    ba2 = ba.reshape(1, 1)
    return pl.pallas_call(
        _ge_kernel,
        out_shape=jax.ShapeDtypeStruct((b, d, n), jnp.float32),
    )(x, weight, bias2, Wq, bq2, Wk, bk2, Wa, ba2)
